# baseline (speedup 1.0000x reference)
import functools

import jax
import jax.numpy as jnp
from jax import lax
from jax.experimental import pallas as pl
from jax.experimental.pallas import tpu as pltpu
from jax.experimental.pallas import tpu_sc as plsc

N = 1024
C = 64
D = 16
NF = 4
CF = C * NF
W = 128
P_PAD = 16
C_OUT = 64

_NC, _NS = 2, 16
_NW = _NC * _NS
_NPW = N // _NW
_RPW = _NPW * D
_JCH = _RPW // 128


def _sc_body(conn_ref, tab_ref, wsc_ref, agg_ref,
             conn_v, idx_v, g_v, gs_v, wsc_v, acc_v,
             sem0, sem1, sem2, sem3, sem_s):
    wid = lax.axis_index("s") * _NC + lax.axis_index("c")
    node0 = wid * _NPW
    pltpu.sync_copy(conn_ref.at[pl.ds(node0, _NPW)], conn_v)
    pltpu.sync_copy(wsc_ref, wsc_v)
    for i in range(_NPW):
        chunk = conn_v[i, pl.ds(0, D)]
        masked = jnp.where(chunk == node0 + i, jnp.int32(N), chunk)
        j, q = divmod(i * D, 128)
        idx_v[j, pl.ds(q, D)] = masked
    sems = [sem0, sem1, sem2, sem3]
    copies = [pltpu.async_copy(
        tab_ref.at[idx_v.at[j]], g_v.at[pl.ds(j * 128, 128)], sems[j])
        for j in range(_JCH)]
    self_cp = pltpu.async_copy(tab_ref.at[pl.ds(node0, _NPW)], gs_v, sem_s)
    self_cp.wait()

    for j in range(_JCH):
        copies[j].wait()
        for kb in range(4):
            wx = [wsc_v[0, pl.ds(16 * (4 * kb + c), 16)] for c in range(4)]
            wy = [wsc_v[1, pl.ds(16 * (4 * kb + c), 16)] for c in range(4)]
            wz = [wsc_v[2, pl.ds(16 * (4 * kb + c), 16)] for c in range(4)]
            wb = [wsc_v[3, pl.ds(16 * (4 * kb + c), 16)] for c in range(4)]
            rb = [jnp.maximum(b, 0.0) for b in wb]

            def node_body(i, _, kb=kb, wx=wx, wy=wy, wz=wz, wb=wb, rb=rb):
                ps = gs_v[i, pl.ds(C, 16)]
                acc = [gs_v[i, pl.ds(16 * c, 16)] * rb[c] for c in range(4)]
                for d in range(D):
                    r = i * D + d
                    dirv = g_v[r, pl.ds(C, 16)] - ps
                    dx, dy, dz = dirv[0], dirv[1], dirv[2]
                    for c in range(4):
                        aff = jnp.maximum(
                            dx * wx[c] + dy * wy[c] + dz * wz[c] + wb[c], 0.0)
                        acc[c] = acc[c] + g_v[r, pl.ds(16 * c, 16)] * aff
                for c in range(4):
                    acc_v[i, pl.ds(16 * (4 * kb + c), 16)] = acc[c]
                return 0

            lax.fori_loop(j * (_NPW // _JCH), (j + 1) * (_NPW // _JCH),
                          node_body, 0)
    pltpu.sync_copy(acc_v, agg_ref.at[pl.ds(node0, _NPW)])


def _tc_body(agg_ref, wf_ref, bf_ref, out_ref):
    out_ref[...] = jnp.maximum(
        jnp.dot(agg_ref[...], wf_ref[...],
                preferred_element_type=jnp.float32) + bf_ref[...], 0.0)


def kernel(node_feats, node_connections, node_positions, Wd, bd, Wf, bf):
    f32 = jnp.float32
    feats = node_feats[0].astype(f32)
    pos = node_positions[0].astype(f32)
    conn = node_connections.astype(jnp.int32)

    pos_tab = jnp.pad(pos, ((0, 0), (0, P_PAD - pos.shape[1])))
    tab = jnp.concatenate(
        [feats, pos_tab, jnp.zeros((N, W - C - P_PAD), f32)], axis=1)
    tab = jnp.concatenate([tab, jnp.zeros((8, W), f32)], axis=0)
    wsc = jnp.concatenate(
        [Wd.T.astype(f32), bd.astype(f32).reshape(1, CF)], axis=0)

    mesh = plsc.VectorSubcoreMesh(core_axis_name="c", subcore_axis_name="s")
    sc_agg = functools.partial(
        pl.kernel, mesh=mesh,
        out_type=jax.ShapeDtypeStruct((N, CF), f32),
        scratch_types=[pltpu.VMEM((_NPW, D), jnp.int32),
                       pltpu.VMEM((_JCH, 128), jnp.int32),
                       pltpu.VMEM((_RPW, W), f32),
                       pltpu.VMEM((_NPW, W), f32),
                       pltpu.VMEM((4, CF), f32),
                       pltpu.VMEM((_NPW, CF), f32),
                       pltpu.SemaphoreType.DMA,
                       pltpu.SemaphoreType.DMA,
                       pltpu.SemaphoreType.DMA,
                       pltpu.SemaphoreType.DMA,
                       pltpu.SemaphoreType.DMA],
    )(_sc_body)
    agg = sc_agg(conn, tab, wsc)

    wf_t = Wf.T.astype(f32)
    bf2 = bf.astype(f32).reshape(1, C_OUT)
    out = pl.pallas_call(
        _tc_body,
        grid=(4,),
        in_specs=[
            pl.BlockSpec((N // 4, CF), lambda i: (i, 0)),
            pl.BlockSpec((CF, C_OUT), lambda i: (0, 0)),
            pl.BlockSpec((1, C_OUT), lambda i: (0, 0)),
        ],
        out_specs=pl.BlockSpec((N // 4, C_OUT), lambda i: (i, 0)),
        out_shape=jax.ShapeDtypeStruct((N, C_OUT), f32),
        compiler_params=pltpu.CompilerParams(
            dimension_semantics=("arbitrary",)),
    )(agg, wf_t, bf2)
    return out[None]


# speedup vs baseline: 1.3892x; 1.3892x over previous
import functools

import jax
import jax.numpy as jnp
from jax import lax
from jax.experimental import pallas as pl
from jax.experimental.pallas import tpu as pltpu
from jax.experimental.pallas import tpu_sc as plsc

N = 1024
C = 64
D = 16
NF = 4
CF = C * NF
W = 128
P_PAD = 16
C_OUT = 64
BLK = 128

_NC, _NS = 2, 16
_NW = _NC * _NS
_RPW = (N * D) // _NW
_JCH = _RPW // 128


def _sc_gather_body(conn_ref, tab_ref, g_out, idx_v, idx2_v, g_v, sem):
    wid = lax.axis_index("s") * _NC + lax.axis_index("c")
    pltpu.sync_copy(conn_ref.at[pl.ds(wid * _JCH, _JCH)], idx_v)
    nbase = (wid % 2) * _RPW
    lanes = lax.iota(jnp.int32, 16)
    for i in range(_RPW // 16):
        j, q = divmod(i, 8)
        chunk = idx_v[j, pl.ds(q * 16, 16)]
        nodes = nbase + i * 16 + lanes
        idx2_v[j, pl.ds(q * 16, 16)] = jnp.where(
            chunk == nodes, jnp.int32(N), chunk)
    copies = []
    for j in range(_JCH):
        copies.append(pltpu.async_copy(
            tab_ref.at[idx2_v.at[j]], g_v.at[pl.ds(j * 128, 128)], sem))
    for cp in copies:
        cp.wait()
    pltpu.sync_copy(g_v, g_out.at[pl.ds(wid * _RPW, _RPW)])


def _tc_body(g_ref, ps_ref, fs_ref, wd_ref, bd_ref, wf_ref, bf_ref, out_ref):
    ps = ps_ref[...]
    wd = wd_ref[...]
    bd = bd_ref[...]
    acc = jnp.zeros((BLK, CF), jnp.float32)
    for d in range(D):
        row = g_ref[d]
        dirv = row[:, C:C + P_PAD] - ps
        aff = jnp.maximum(
            jnp.dot(dirv, wd, preferred_element_type=jnp.float32) + bd, 0.0)
        fg = row[:, :C]
        stacked = jnp.concatenate([fg] * NF, axis=1)
        acc = acc + stacked * aff
    fs = fs_ref[...]
    acc = acc + jnp.concatenate([fs] * NF, axis=1) * jnp.maximum(bd, 0.0)
    out = jnp.maximum(
        jnp.dot(acc, wf_ref[...], preferred_element_type=jnp.float32)
        + bf_ref[...], 0.0)
    out_ref[...] = out


def kernel(node_feats, node_connections, node_positions, Wd, bd, Wf, bf):
    f32 = jnp.float32
    feats = node_feats[0].astype(f32)
    pos = node_positions[0].astype(f32)
    conn = node_connections.astype(jnp.int32)

    pos_tab = jnp.pad(pos, ((0, 0), (0, P_PAD - pos.shape[1])))
    tab = jnp.concatenate(
        [feats, pos_tab, jnp.zeros((N, W - C - P_PAD), f32)], axis=1)
    tab = jnp.concatenate([tab, jnp.zeros((8, W), f32)], axis=0)
    conn_dmaj = conn.T.reshape(-1, 128)

    mesh = plsc.VectorSubcoreMesh(core_axis_name="c", subcore_axis_name="s")
    sc_gather = functools.partial(
        pl.kernel, mesh=mesh,
        out_type=jax.ShapeDtypeStruct((N * D, W), f32),
        scratch_types=[pltpu.VMEM((_JCH, 128), jnp.int32),
                       pltpu.VMEM((_JCH, 128), jnp.int32),
                       pltpu.VMEM((_RPW, W), f32),
                       pltpu.SemaphoreType.DMA],
    )(_sc_gather_body)
    g = sc_gather(conn_dmaj, tab)

    g3 = g.reshape(D, N, W)
    wd_p = jnp.pad(Wd.T.astype(f32), ((0, P_PAD - Wd.shape[1]), (0, 0)))
    bd2 = bd.astype(f32).reshape(1, CF)
    wf_t = Wf.T.astype(f32)
    bf2 = bf.astype(f32).reshape(1, C_OUT)

    out = pl.pallas_call(
        _tc_body,
        grid=(N // BLK,),
        in_specs=[
            pl.BlockSpec((D, BLK, W), lambda i: (0, i, 0)),
            pl.BlockSpec((BLK, P_PAD), lambda i: (i, 0)),
            pl.BlockSpec((BLK, C), lambda i: (i, 0)),
            pl.BlockSpec((P_PAD, CF), lambda i: (0, 0)),
            pl.BlockSpec((1, CF), lambda i: (0, 0)),
            pl.BlockSpec((CF, C_OUT), lambda i: (0, 0)),
            pl.BlockSpec((1, C_OUT), lambda i: (0, 0)),
        ],
        out_specs=pl.BlockSpec((BLK, C_OUT), lambda i: (i, 0)),
        out_shape=jax.ShapeDtypeStruct((N, C_OUT), f32),
        compiler_params=pltpu.CompilerParams(
            dimension_semantics=("arbitrary",)),
    )(g3, pos_tab, feats, wd_p, bd2, wf_t, bf2)
    return out[None]
